# Initial kernel scaffold; baseline (speedup 1.0000x reference)
import jax
import jax.numpy as jnp
from jax.experimental import pallas as pl


def kernel(x, edge_index, W1, a_src1, a_dst1, b1, W2, a_src2, a_dst2, b2, W3, a_src3, a_dst3, b3):
    raise NotImplementedError("write your pallas kernel here")


# speedup vs baseline: 48.4078x; 48.4078x over previous
import functools

import jax
import jax.numpy as jnp
from jax import lax
from jax.experimental import pallas as pl
from jax.experimental.pallas import tpu as pltpu
from jax.experimental.pallas import tpu_sc as plsc

N = 10000
E = 320000
D_IN = 128
HEADS = 8
DIM = 8
HID = 64
NCLS = 40
FW3 = 48
NP = 10240
HL = 16
CHUNK = 128
NTILES = 32
EPT = 10112
NCHUNK = EPT // CHUNK
EP = NTILES * EPT
TRASH = NP - 1
RPT = NP // 16

f32 = jnp.float32
_BN = 256


def _dense1_body(x_ref, w_ref, asw_ref, adw_ref, h_ref, as_ref, ad_ref):
    h = jnp.dot(x_ref[...], w_ref[...], preferred_element_type=f32)
    h_ref[...] = h
    as_ref[...] = jnp.dot(h, asw_ref[...], preferred_element_type=f32)
    ad_ref[...] = jnp.dot(h, adw_ref[...], preferred_element_type=f32)


def _dense1(x, w, asw, adw):
    return pl.pallas_call(
        _dense1_body,
        grid=(NP // _BN,),
        in_specs=[
            pl.BlockSpec((_BN, D_IN), lambda i: (i, 0)),
            pl.BlockSpec((D_IN, HID), lambda i: (0, 0)),
            pl.BlockSpec((HID, HL), lambda i: (0, 0)),
            pl.BlockSpec((HID, HL), lambda i: (0, 0)),
        ],
        out_specs=[
            pl.BlockSpec((_BN, HID), lambda i: (i, 0)),
            pl.BlockSpec((_BN, HL), lambda i: (i, 0)),
            pl.BlockSpec((_BN, HL), lambda i: (i, 0)),
        ],
        out_shape=[
            jax.ShapeDtypeStruct((NP, HID), f32),
            jax.ShapeDtypeStruct((NP, HL), f32),
            jax.ShapeDtypeStruct((NP, HL), f32),
        ],
    )(x, w, asw, adw)


def _dense_mid_body(p0_ref, p1_ref, b_ref, w_ref, asw_ref, adw_ref,
                    h_ref, as_ref, ad_ref):
    o = p0_ref[...] + p1_ref[...] + b_ref[...]
    e = jnp.where(o > 0, o, jnp.exp(jnp.minimum(o, 0.0)) - 1.0)
    h = jnp.dot(e, w_ref[...], preferred_element_type=f32)
    h_ref[...] = h
    as_ref[...] = jnp.dot(h, asw_ref[...], preferred_element_type=f32)
    ad_ref[...] = jnp.dot(h, adw_ref[...], preferred_element_type=f32)


def _dense_mid(p0, p1, brow, w, asw, adw):
    fo = w.shape[1]
    return pl.pallas_call(
        _dense_mid_body,
        grid=(NP // _BN,),
        in_specs=[
            pl.BlockSpec((_BN, HID), lambda i: (i, 0)),
            pl.BlockSpec((_BN, HID), lambda i: (i, 0)),
            pl.BlockSpec((1, HID), lambda i: (0, 0)),
            pl.BlockSpec((HID, fo), lambda i: (0, 0)),
            pl.BlockSpec((fo, HL), lambda i: (0, 0)),
            pl.BlockSpec((fo, HL), lambda i: (0, 0)),
        ],
        out_specs=[
            pl.BlockSpec((_BN, fo), lambda i: (i, 0)),
            pl.BlockSpec((_BN, HL), lambda i: (i, 0)),
            pl.BlockSpec((_BN, HL), lambda i: (i, 0)),
        ],
        out_shape=[
            jax.ShapeDtypeStruct((NP, fo), f32),
            jax.ShapeDtypeStruct((NP, HL), f32),
            jax.ShapeDtypeStruct((NP, HL), f32),
        ],
    )(p0, p1, brow, w, asw, adw)


def _final_body(p0_ref, p1_ref, b_ref, o_ref):
    z = p0_ref[...] + p1_ref[...] + b_ref[...]
    col = lax.broadcasted_iota(jnp.int32, z.shape, 1)
    mask = col < NCLS
    zm = jnp.where(mask, z, -1e30)
    m = jnp.max(zm, axis=1, keepdims=True)
    ez = jnp.where(mask, jnp.exp(z - m), 0.0)
    ssum = jnp.sum(ez, axis=1, keepdims=True)
    o_ref[...] = z - m - jnp.log(ssum)


def _final(p0, p1, brow):
    return pl.pallas_call(
        _final_body,
        grid=(NP // _BN,),
        in_specs=[
            pl.BlockSpec((_BN, FW3), lambda i: (i, 0)),
            pl.BlockSpec((_BN, FW3), lambda i: (i, 0)),
            pl.BlockSpec((1, FW3), lambda i: (0, 0)),
        ],
        out_specs=pl.BlockSpec((_BN, FW3), lambda i: (i, 0)),
        out_shape=jax.ShapeDtypeStruct((NP, FW3), f32),
    )(p0, p1, brow)


def _make_sca():
    mesh = plsc.VectorSubcoreMesh(core_axis_name="c", subcore_axis_name="s")

    @functools.partial(
        pl.kernel,
        out_type=[
            jax.ShapeDtypeStruct((EP, HL), f32),
            jax.ShapeDtypeStruct((2, NP, HL), f32),
        ],
        scratch_types=[
            pltpu.VMEM((CHUNK,), jnp.int32),
            pltpu.VMEM((CHUNK,), jnp.int32),
            pltpu.VMEM((CHUNK, HL), f32),
            pltpu.VMEM((CHUNK, HL), f32),
            pltpu.VMEM((CHUNK, HL), f32),
            pltpu.VMEM((RPT, HL), f32),
            pltpu.VMEM_SHARED((NP, HL), f32),
            pltpu.SemaphoreType.DMA,
            pltpu.SemaphoreType.DMA,
        ],
        mesh=mesh,
        compiler_params=pltpu.CompilerParams(use_tc_tiling_on_sc=False),
    )
    def sca(srcp, dstp, alsrc, aldst, ex_out, den_out,
            idx_s, idx_d, asv, adv, exv, dumpv, den_sp, sem0, sem1):
        c = lax.axis_index("c")
        s = lax.axis_index("s")
        wid = c * 16 + s
        for r in range(CHUNK):
            exv[r] = jnp.zeros((HL,), f32)
        row0 = s * RPT
        for k in range(RPT // CHUNK):
            pltpu.sync_copy(exv, den_sp.at[pl.ds(row0 + k * CHUNK, CHUNK)])
        plsc.subcore_barrier()

        base_e = wid * EPT

        def body(i, carry):
            off = base_e + i * CHUNK
            pltpu.sync_copy(srcp.at[pl.ds(off, CHUNK)], idx_s)
            pltpu.sync_copy(dstp.at[pl.ds(off, CHUNK)], idx_d)
            cp0 = pltpu.async_copy(alsrc.at[idx_s], asv, sem0)
            cp1 = pltpu.async_copy(aldst.at[idx_d], adv, sem1)
            cp0.wait()
            cp1.wait()

            def cbody(j, c2):
                for u in range(8):
                    r = j * 8 + u
                    v = asv[r] + adv[r]
                    v = jnp.maximum(v, 0.2 * v)
                    exv[r] = jnp.exp(v)
                return c2

            lax.fori_loop(0, CHUNK // 8, cbody, 0)
            pltpu.sync_copy(exv, ex_out.at[pl.ds(off, CHUNK)])
            pltpu.sync_copy(exv, den_sp.at[idx_d], add=True)
            return carry

        lax.fori_loop(0, NCHUNK, body, 0)
        plsc.subcore_barrier()
        pltpu.sync_copy(den_sp.at[pl.ds(row0, RPT)], dumpv)
        pltpu.sync_copy(dumpv, den_out.at[c].at[pl.ds(row0, RPT)])

    return sca


def _make_scb(fw):
    nseg = fw // 16
    mesh = plsc.VectorSubcoreMesh(core_axis_name="c", subcore_axis_name="s")

    @functools.partial(
        pl.kernel,
        out_type=[jax.ShapeDtypeStruct((2, NP, fw), f32)],
        scratch_types=[
            pltpu.VMEM((CHUNK,), jnp.int32),
            pltpu.VMEM((CHUNK,), jnp.int32),
            pltpu.VMEM((CHUNK, fw), f32),
            pltpu.VMEM((CHUNK, HL), f32),
            pltpu.VMEM((CHUNK, HL), f32),
            pltpu.VMEM((CHUNK, HL), f32),
            pltpu.VMEM((CHUNK, fw), f32),
            pltpu.VMEM((RPT, fw), f32),
            pltpu.VMEM_SHARED((NP, fw), f32),
            pltpu.SemaphoreType.DMA,
            pltpu.SemaphoreType.DMA,
            pltpu.SemaphoreType.DMA,
        ],
        mesh=mesh,
        compiler_params=pltpu.CompilerParams(use_tc_tiling_on_sc=False),
    )
    def scb(srcp, dstp, ex_in, den0, den1, h_in, out_hbm,
            idx_s, idx_d, hv, d0v, d1v, exv, msgv, dumpv, out_sp,
            sem0, sem1, sem2):
        c = lax.axis_index("c")
        s = lax.axis_index("s")
        wid = c * 16 + s
        for r in range(CHUNK):
            for v in range(nseg):
                msgv[r, pl.ds(v * 16, 16)] = jnp.zeros((16,), f32)
        row0 = s * RPT
        for k in range(RPT // CHUNK):
            pltpu.sync_copy(msgv, out_sp.at[pl.ds(row0 + k * CHUNK, CHUNK)])
        plsc.subcore_barrier()

        base_e = wid * EPT

        def body(i, carry):
            off = base_e + i * CHUNK
            pltpu.sync_copy(srcp.at[pl.ds(off, CHUNK)], idx_s)
            pltpu.sync_copy(dstp.at[pl.ds(off, CHUNK)], idx_d)
            cp0 = pltpu.async_copy(h_in.at[idx_s], hv, sem0)
            cp1 = pltpu.async_copy(den0.at[idx_d], d0v, sem1)
            cp2 = pltpu.async_copy(den1.at[idx_d], d1v, sem2)
            pltpu.sync_copy(ex_in.at[pl.ds(off, CHUNK)], exv)
            cp0.wait()
            cp1.wait()
            cp2.wait()

            def rbody(j, c2):
                for u in range(8):
                    r = j * 8 + u
                    exv[r] = exv[r] / (d0v[r] + d1v[r] + 1e-16)
                return c2

            lax.fori_loop(0, CHUNK // 8, rbody, 0)

            lo_mask = lax.iota(jnp.int32, 16) < 8

            def mbody(j, c2):
                for u in range(4):
                    e = j * 4 + u
                    arow = exv[e]
                    for v in range(nseg):
                        seg = pl.ds(v * 16, 16)
                        if fw == HID:
                            a16 = jnp.where(lo_mask, arow[2 * v],
                                            arow[2 * v + 1])
                        else:
                            a16 = jnp.where(lo_mask, arow[0], arow[0])
                        msgv[e, seg] = hv[e, seg] * a16
                return c2

            lax.fori_loop(0, CHUNK // 4, mbody, 0)
            pltpu.sync_copy(msgv, out_sp.at[idx_d], add=True)
            return carry

        lax.fori_loop(0, NCHUNK, body, 0)
        plsc.subcore_barrier()
        pltpu.sync_copy(out_sp.at[pl.ds(row0, RPT)], dumpv)
        pltpu.sync_copy(dumpv, out_hbm.at[c].at[pl.ds(row0, RPT)])

    return scb


_SCA = _make_sca()
_SCB64 = _make_scb(HID)
_SCB48 = _make_scb(FW3)


def _expand_att(a):
    ar = jnp.arange(HID)
    return jnp.zeros((HID, HL), f32).at[ar, ar // DIM].set(a.reshape(HID))


def kernel(x, edge_index, W1, a_src1, a_dst1, b1, W2, a_src2, a_dst2, b2,
           W3, a_src3, a_dst3, b3):
    xp = jnp.zeros((NP, D_IN), f32).at[:N].set(x)
    eper = E // NTILES
    src = edge_index[0].reshape(NTILES, eper)
    dst = edge_index[1].reshape(NTILES, eper)
    padw = ((0, 0), (0, EPT - eper))
    srcp = jnp.pad(src, padw, constant_values=TRASH).reshape(EP)
    dstp = jnp.pad(dst, padw, constant_values=TRASH).reshape(EP)

    As1, Ad1 = _expand_att(a_src1), _expand_att(a_dst1)
    As2, Ad2 = _expand_att(a_src2), _expand_att(a_dst2)
    a3s = jnp.zeros((FW3,), f32).at[:NCLS].set(a_src3[0])
    a3d = jnp.zeros((FW3,), f32).at[:NCLS].set(a_dst3[0])
    As3 = jnp.tile(a3s[:, None], (1, HL))
    Ad3 = jnp.tile(a3d[:, None], (1, HL))
    W3p = jnp.zeros((HID, FW3), f32).at[:, :NCLS].set(W3)
    b3p = jnp.zeros((FW3,), f32).at[:NCLS].set(b3)

    h1, as1, ad1 = _dense1(xp, W1, As1, Ad1)
    ex1, den1 = _SCA(srcp, dstp, as1, ad1)
    (outp1,) = _SCB64(srcp, dstp, ex1, den1[0], den1[1], h1)
    h2, as2, ad2 = _dense_mid(outp1[0], outp1[1], b1.reshape(1, HID),
                              W2, As2, Ad2)
    ex2, den2 = _SCA(srcp, dstp, as2, ad2)
    (outp2,) = _SCB64(srcp, dstp, ex2, den2[0], den2[1], h2)
    h3, as3, ad3 = _dense_mid(outp2[0], outp2[1], b2.reshape(1, HID),
                              W3p, As3, Ad3)
    ex3, den3 = _SCA(srcp, dstp, as3, ad3)
    (outp3,) = _SCB48(srcp, dstp, ex3, den3[0], den3[1], h3)
    o = _final(outp3[0], outp3[1], b3p.reshape(1, FW3))
    return o[:N, :NCLS]


# speedup vs baseline: 52.1164x; 1.0766x over previous
import functools

import jax
import jax.numpy as jnp
from jax import lax
from jax.experimental import pallas as pl
from jax.experimental.pallas import tpu as pltpu
from jax.experimental.pallas import tpu_sc as plsc

N = 10000
E = 320000
D_IN = 128
HEADS = 8
DIM = 8
HID = 64
NCLS = 40
FW3 = 48
NP = 10240
HL = 16
CHUNK = 128
NTILES = 32
EPT = 10240
NCHUNK = EPT // CHUNK
EP = NTILES * EPT
TRASH = NP - 1
RPT = NP // 16

f32 = jnp.float32
_BN = 256


def _dense1_body(x_ref, w_ref, asw_ref, adw_ref, h_ref, as_ref, ad_ref):
    h = jnp.dot(x_ref[...], w_ref[...], preferred_element_type=f32)
    h_ref[...] = h
    as_ref[...] = jnp.dot(h, asw_ref[...], preferred_element_type=f32)
    ad_ref[...] = jnp.dot(h, adw_ref[...], preferred_element_type=f32)


def _dense1(x, w, asw, adw):
    return pl.pallas_call(
        _dense1_body,
        grid=(NP // _BN,),
        in_specs=[
            pl.BlockSpec((_BN, D_IN), lambda i: (i, 0)),
            pl.BlockSpec((D_IN, HID), lambda i: (0, 0)),
            pl.BlockSpec((HID, HL), lambda i: (0, 0)),
            pl.BlockSpec((HID, HL), lambda i: (0, 0)),
        ],
        out_specs=[
            pl.BlockSpec((_BN, HID), lambda i: (i, 0)),
            pl.BlockSpec((_BN, HL), lambda i: (i, 0)),
            pl.BlockSpec((_BN, HL), lambda i: (i, 0)),
        ],
        out_shape=[
            jax.ShapeDtypeStruct((NP, HID), f32),
            jax.ShapeDtypeStruct((NP, HL), f32),
            jax.ShapeDtypeStruct((NP, HL), f32),
        ],
    )(x, w, asw, adw)


def _dense_mid_body(p0_ref, p1_ref, b_ref, w_ref, asw_ref, adw_ref,
                    h_ref, as_ref, ad_ref):
    o = p0_ref[...] + p1_ref[...] + b_ref[...]
    e = jnp.where(o > 0, o, jnp.exp(jnp.minimum(o, 0.0)) - 1.0)
    h = jnp.dot(e, w_ref[...], preferred_element_type=f32)
    h_ref[...] = h
    as_ref[...] = jnp.dot(h, asw_ref[...], preferred_element_type=f32)
    ad_ref[...] = jnp.dot(h, adw_ref[...], preferred_element_type=f32)


def _dense_mid(p0, p1, brow, w, asw, adw):
    fo = w.shape[1]
    return pl.pallas_call(
        _dense_mid_body,
        grid=(NP // _BN,),
        in_specs=[
            pl.BlockSpec((_BN, HID), lambda i: (i, 0)),
            pl.BlockSpec((_BN, HID), lambda i: (i, 0)),
            pl.BlockSpec((1, HID), lambda i: (0, 0)),
            pl.BlockSpec((HID, fo), lambda i: (0, 0)),
            pl.BlockSpec((fo, HL), lambda i: (0, 0)),
            pl.BlockSpec((fo, HL), lambda i: (0, 0)),
        ],
        out_specs=[
            pl.BlockSpec((_BN, fo), lambda i: (i, 0)),
            pl.BlockSpec((_BN, HL), lambda i: (i, 0)),
            pl.BlockSpec((_BN, HL), lambda i: (i, 0)),
        ],
        out_shape=[
            jax.ShapeDtypeStruct((NP, fo), f32),
            jax.ShapeDtypeStruct((NP, HL), f32),
            jax.ShapeDtypeStruct((NP, HL), f32),
        ],
    )(p0, p1, brow, w, asw, adw)


def _final_body(p0_ref, p1_ref, b_ref, o_ref):
    z = p0_ref[...] + p1_ref[...] + b_ref[...]
    col = lax.broadcasted_iota(jnp.int32, z.shape, 1)
    mask = col < NCLS
    zm = jnp.where(mask, z, -1e30)
    m = jnp.max(zm, axis=1, keepdims=True)
    ez = jnp.where(mask, jnp.exp(z - m), 0.0)
    ssum = jnp.sum(ez, axis=1, keepdims=True)
    o_ref[...] = z - m - jnp.log(ssum)


def _final(p0, p1, brow):
    return pl.pallas_call(
        _final_body,
        grid=(NP // _BN,),
        in_specs=[
            pl.BlockSpec((_BN, FW3), lambda i: (i, 0)),
            pl.BlockSpec((_BN, FW3), lambda i: (i, 0)),
            pl.BlockSpec((1, FW3), lambda i: (0, 0)),
        ],
        out_specs=pl.BlockSpec((_BN, FW3), lambda i: (i, 0)),
        out_shape=jax.ShapeDtypeStruct((NP, FW3), f32),
    )(p0, p1, brow)


def _make_sca():
    mesh = plsc.VectorSubcoreMesh(core_axis_name="c", subcore_axis_name="s")

    @functools.partial(
        pl.kernel,
        out_type=[
            jax.ShapeDtypeStruct((EP, HL), f32),
            jax.ShapeDtypeStruct((2, NP, HL), f32),
        ],
        scratch_types=[
            pltpu.VMEM((2, CHUNK), jnp.int32),
            pltpu.VMEM((2, CHUNK), jnp.int32),
            pltpu.VMEM((2, CHUNK, HL), f32),
            pltpu.VMEM((2, CHUNK, HL), f32),
            pltpu.VMEM((2, CHUNK, HL), f32),
            pltpu.VMEM((RPT, HL), f32),
            pltpu.VMEM_SHARED((NP, HL), f32),
            pltpu.SemaphoreType.DMA,
            pltpu.SemaphoreType.DMA,
            pltpu.SemaphoreType.DMA,
            pltpu.SemaphoreType.DMA,
        ],
        mesh=mesh,
        compiler_params=pltpu.CompilerParams(use_tc_tiling_on_sc=False),
    )
    def sca(srcp, dstp, alsrc, aldst, ex_out, den_out,
            idx_s, idx_d, asv, adv, exv, dumpv, den_sp,
            sas0, sas1, sad0, sad1):
        c = lax.axis_index("c")
        s = lax.axis_index("s")
        wid = c * 16 + s
        base_e = wid * EPT
        sas = (sas0, sas1)
        sad = (sad0, sad1)

        def prefetch(off, b):
            pltpu.sync_copy(srcp.at[pl.ds(off, CHUNK)], idx_s.at[b])
            pltpu.sync_copy(dstp.at[pl.ds(off, CHUNK)], idx_d.at[b])
            pltpu.async_copy(alsrc.at[idx_s.at[b]], asv.at[b], sas[b])
            pltpu.async_copy(aldst.at[idx_d.at[b]], adv.at[b], sad[b])

        prefetch(base_e, 0)
        for r in range(CHUNK):
            dumpv[r] = jnp.zeros((HL,), f32)
        row0 = s * RPT
        for k in range(RPT // CHUNK):
            pltpu.sync_copy(dumpv.at[pl.ds(0, CHUNK)],
                            den_sp.at[pl.ds(row0 + k * CHUNK, CHUNK)])
        plsc.subcore_barrier()

        def body(kk, carry):
            for b in range(2):
                i = kk * 2 + b
                nb = 1 - b

                @pl.when(i < NCHUNK - 1)
                def _():
                    prefetch(base_e + (i + 1) * CHUNK, nb)

                pltpu.make_async_copy(
                    alsrc.at[pl.ds(0, CHUNK)], asv.at[b], sas[b]).wait()
                pltpu.make_async_copy(
                    aldst.at[pl.ds(0, CHUNK)], adv.at[b], sad[b]).wait()

                def cbody(j, c2):
                    for u in range(8):
                        r = j * 8 + u
                        v = asv[b, r] + adv[b, r]
                        v = jnp.maximum(v, 0.2 * v)
                        exv[b, r] = jnp.exp(v)
                    return c2

                lax.fori_loop(0, CHUNK // 8, cbody, 0)
                off = base_e + i * CHUNK
                pltpu.sync_copy(exv.at[b], ex_out.at[pl.ds(off, CHUNK)])
                pltpu.sync_copy(exv.at[b], den_sp.at[idx_d.at[b]], add=True)
            return carry

        lax.fori_loop(0, NCHUNK // 2, body, 0)
        plsc.subcore_barrier()
        pltpu.sync_copy(den_sp.at[pl.ds(row0, RPT)], dumpv)
        pltpu.sync_copy(dumpv, den_out.at[c].at[pl.ds(row0, RPT)])

    return sca


def _make_scb(fw):
    nseg = fw // 16
    mesh = plsc.VectorSubcoreMesh(core_axis_name="c", subcore_axis_name="s")

    @functools.partial(
        pl.kernel,
        out_type=[jax.ShapeDtypeStruct((2, NP, fw), f32)],
        scratch_types=[
            pltpu.VMEM((2, CHUNK), jnp.int32),
            pltpu.VMEM((2, CHUNK), jnp.int32),
            pltpu.VMEM((2, CHUNK, fw), f32),
            pltpu.VMEM((2, CHUNK, HL), f32),
            pltpu.VMEM((2, CHUNK, HL), f32),
            pltpu.VMEM((2, CHUNK, HL), f32),
            pltpu.VMEM((2, CHUNK, fw), f32),
            pltpu.VMEM((RPT, fw), f32),
            pltpu.VMEM_SHARED((NP, fw), f32),
            pltpu.SemaphoreType.DMA,
            pltpu.SemaphoreType.DMA,
            pltpu.SemaphoreType.DMA,
            pltpu.SemaphoreType.DMA,
            pltpu.SemaphoreType.DMA,
            pltpu.SemaphoreType.DMA,
            pltpu.SemaphoreType.DMA,
            pltpu.SemaphoreType.DMA,
        ],
        mesh=mesh,
        compiler_params=pltpu.CompilerParams(use_tc_tiling_on_sc=False),
    )
    def scb(srcp, dstp, ex_in, den0, den1, h_in, out_hbm,
            idx_s, idx_d, hv, d0v, d1v, exv, msgv, dumpv, out_sp,
            sh0, sh1, s00, s01, s10, s11, se0, se1):
        c = lax.axis_index("c")
        s = lax.axis_index("s")
        wid = c * 16 + s
        base_e = wid * EPT
        sh = (sh0, sh1)
        s0 = (s00, s01)
        s1 = (s10, s11)
        se = (se0, se1)

        def prefetch(off, b):
            pltpu.sync_copy(srcp.at[pl.ds(off, CHUNK)], idx_s.at[b])
            pltpu.sync_copy(dstp.at[pl.ds(off, CHUNK)], idx_d.at[b])
            pltpu.async_copy(h_in.at[idx_s.at[b]], hv.at[b], sh[b])
            pltpu.async_copy(den0.at[idx_d.at[b]], d0v.at[b], s0[b])
            pltpu.async_copy(den1.at[idx_d.at[b]], d1v.at[b], s1[b])
            pltpu.async_copy(ex_in.at[pl.ds(off, CHUNK)], exv.at[b], se[b])

        prefetch(base_e, 0)
        for r in range(CHUNK):
            for v in range(nseg):
                dumpv[r, pl.ds(v * 16, 16)] = jnp.zeros((16,), f32)
        row0 = s * RPT
        for k in range(RPT // CHUNK):
            pltpu.sync_copy(dumpv.at[pl.ds(0, CHUNK)],
                            out_sp.at[pl.ds(row0 + k * CHUNK, CHUNK)])
        plsc.subcore_barrier()

        lo_mask = lax.iota(jnp.int32, 16) < 8

        def body(kk, carry):
            for b in range(2):
                i = kk * 2 + b
                nb = 1 - b

                @pl.when(i < NCHUNK - 1)
                def _():
                    prefetch(base_e + (i + 1) * CHUNK, nb)

                pltpu.make_async_copy(
                    h_in.at[pl.ds(0, CHUNK)], hv.at[b], sh[b]).wait()
                pltpu.make_async_copy(
                    den0.at[pl.ds(0, CHUNK)], d0v.at[b], s0[b]).wait()
                pltpu.make_async_copy(
                    den1.at[pl.ds(0, CHUNK)], d1v.at[b], s1[b]).wait()
                pltpu.make_async_copy(
                    ex_in.at[pl.ds(0, CHUNK)], exv.at[b], se[b]).wait()

                def rbody(j, c2):
                    for u in range(8):
                        r = j * 8 + u
                        exv[b, r] = exv[b, r] / (
                            d0v[b, r] + d1v[b, r] + 1e-16)
                    return c2

                lax.fori_loop(0, CHUNK // 8, rbody, 0)

                def mbody(j, c2):
                    for u in range(4):
                        e = j * 4 + u
                        arow = exv[b, e]
                        for v in range(nseg):
                            seg = pl.ds(v * 16, 16)
                            if fw == HID:
                                a16 = jnp.where(lo_mask, arow[2 * v],
                                                arow[2 * v + 1])
                            else:
                                a16 = jnp.where(lo_mask, arow[0], arow[0])
                            msgv[b, e, seg] = hv[b, e, seg] * a16
                    return c2

                lax.fori_loop(0, CHUNK // 4, mbody, 0)
                pltpu.sync_copy(msgv.at[b], out_sp.at[idx_d.at[b]],
                                add=True)
            return carry

        lax.fori_loop(0, NCHUNK // 2, body, 0)
        plsc.subcore_barrier()
        pltpu.sync_copy(out_sp.at[pl.ds(row0, RPT)], dumpv)
        pltpu.sync_copy(dumpv, out_hbm.at[c].at[pl.ds(row0, RPT)])

    return scb


_SCA = _make_sca()
_SCB64 = _make_scb(HID)
_SCB48 = _make_scb(FW3)


def _expand_att(a):
    ar = jnp.arange(HID)
    return jnp.zeros((HID, HL), f32).at[ar, ar // DIM].set(a.reshape(HID))


def kernel(x, edge_index, W1, a_src1, a_dst1, b1, W2, a_src2, a_dst2, b2,
           W3, a_src3, a_dst3, b3):
    xp = jnp.zeros((NP, D_IN), f32).at[:N].set(x)
    eper = E // NTILES
    src = edge_index[0].reshape(NTILES, eper)
    dst = edge_index[1].reshape(NTILES, eper)
    padw = ((0, 0), (0, EPT - eper))
    srcp = jnp.pad(src, padw, constant_values=TRASH).reshape(EP)
    dstp = jnp.pad(dst, padw, constant_values=TRASH).reshape(EP)

    As1, Ad1 = _expand_att(a_src1), _expand_att(a_dst1)
    As2, Ad2 = _expand_att(a_src2), _expand_att(a_dst2)
    a3s = jnp.zeros((FW3,), f32).at[:NCLS].set(a_src3[0])
    a3d = jnp.zeros((FW3,), f32).at[:NCLS].set(a_dst3[0])
    As3 = jnp.tile(a3s[:, None], (1, HL))
    Ad3 = jnp.tile(a3d[:, None], (1, HL))
    W3p = jnp.zeros((HID, FW3), f32).at[:, :NCLS].set(W3)
    b3p = jnp.zeros((FW3,), f32).at[:NCLS].set(b3)

    h1, as1, ad1 = _dense1(xp, W1, As1, Ad1)
    ex1, den1 = _SCA(srcp, dstp, as1, ad1)
    (outp1,) = _SCB64(srcp, dstp, ex1, den1[0], den1[1], h1)
    h2, as2, ad2 = _dense_mid(outp1[0], outp1[1], b1.reshape(1, HID),
                              W2, As2, Ad2)
    ex2, den2 = _SCA(srcp, dstp, as2, ad2)
    (outp2,) = _SCB64(srcp, dstp, ex2, den2[0], den2[1], h2)
    h3, as3, ad3 = _dense_mid(outp2[0], outp2[1], b2.reshape(1, HID),
                              W3p, As3, Ad3)
    ex3, den3 = _SCA(srcp, dstp, as3, ad3)
    (outp3,) = _SCB48(srcp, dstp, ex3, den3[0], den3[1], h3)
    o = _final(outp3[0], outp3[1], b3p.reshape(1, FW3))
    return o[:N, :NCLS]


# speedup vs baseline: 65.5238x; 1.2573x over previous
import functools

import jax
import jax.numpy as jnp
from jax import lax
from jax.experimental import pallas as pl
from jax.experimental.pallas import tpu as pltpu
from jax.experimental.pallas import tpu_sc as plsc

N = 10000
E = 320000
D_IN = 128
HEADS = 8
DIM = 8
HID = 64
NCLS = 40
FW3 = 48
NP = 10240
HL = 16
CHUNK = 128
NTILES = 32
EPT = 10240
NCHUNK = EPT // CHUNK
EP = NTILES * EPT
TRASH = NP - 1
RPT = NP // 16

f32 = jnp.float32
_BN = 256


def _dense1_body(x_ref, w_ref, asw_ref, adw_ref, h_ref, as_ref, ad_ref):
    h = jnp.dot(x_ref[...], w_ref[...], preferred_element_type=f32)
    h_ref[...] = h
    as_ref[...] = jnp.dot(h, asw_ref[...], preferred_element_type=f32)
    ad_ref[...] = jnp.dot(h, adw_ref[...], preferred_element_type=f32)


def _dense1(x, w, asw, adw):
    return pl.pallas_call(
        _dense1_body,
        grid=(NP // _BN,),
        in_specs=[
            pl.BlockSpec((_BN, D_IN), lambda i: (i, 0)),
            pl.BlockSpec((D_IN, HID), lambda i: (0, 0)),
            pl.BlockSpec((HID, HL), lambda i: (0, 0)),
            pl.BlockSpec((HID, HL), lambda i: (0, 0)),
        ],
        out_specs=[
            pl.BlockSpec((_BN, HID), lambda i: (i, 0)),
            pl.BlockSpec((_BN, HL), lambda i: (i, 0)),
            pl.BlockSpec((_BN, HL), lambda i: (i, 0)),
        ],
        out_shape=[
            jax.ShapeDtypeStruct((NP, HID), f32),
            jax.ShapeDtypeStruct((NP, HL), f32),
            jax.ShapeDtypeStruct((NP, HL), f32),
        ],
    )(x, w, asw, adw)


def _dense_mid_body(p0_ref, p1_ref, b_ref, w_ref, asw_ref, adw_ref,
                    h_ref, as_ref, ad_ref):
    o = p0_ref[...] + p1_ref[...] + b_ref[...]
    e = jnp.where(o > 0, o, jnp.exp(jnp.minimum(o, 0.0)) - 1.0)
    h = jnp.dot(e, w_ref[...], preferred_element_type=f32)
    h_ref[...] = h
    as_ref[...] = jnp.dot(h, asw_ref[...], preferred_element_type=f32)
    ad_ref[...] = jnp.dot(h, adw_ref[...], preferred_element_type=f32)


def _dense_mid(p0, p1, brow, w, asw, adw):
    fo = w.shape[1]
    return pl.pallas_call(
        _dense_mid_body,
        grid=(NP // _BN,),
        in_specs=[
            pl.BlockSpec((_BN, HID), lambda i: (i, 0)),
            pl.BlockSpec((_BN, HID), lambda i: (i, 0)),
            pl.BlockSpec((1, HID), lambda i: (0, 0)),
            pl.BlockSpec((HID, fo), lambda i: (0, 0)),
            pl.BlockSpec((fo, HL), lambda i: (0, 0)),
            pl.BlockSpec((fo, HL), lambda i: (0, 0)),
        ],
        out_specs=[
            pl.BlockSpec((_BN, fo), lambda i: (i, 0)),
            pl.BlockSpec((_BN, HL), lambda i: (i, 0)),
            pl.BlockSpec((_BN, HL), lambda i: (i, 0)),
        ],
        out_shape=[
            jax.ShapeDtypeStruct((NP, fo), f32),
            jax.ShapeDtypeStruct((NP, HL), f32),
            jax.ShapeDtypeStruct((NP, HL), f32),
        ],
    )(p0, p1, brow, w, asw, adw)


def _final_body(p0_ref, p1_ref, b_ref, o_ref):
    z = p0_ref[...] + p1_ref[...] + b_ref[...]
    col = lax.broadcasted_iota(jnp.int32, z.shape, 1)
    mask = col < NCLS
    zm = jnp.where(mask, z, -1e30)
    m = jnp.max(zm, axis=1, keepdims=True)
    ez = jnp.where(mask, jnp.exp(z - m), 0.0)
    ssum = jnp.sum(ez, axis=1, keepdims=True)
    o_ref[...] = z - m - jnp.log(ssum)


def _final(p0, p1, brow):
    return pl.pallas_call(
        _final_body,
        grid=(NP // _BN,),
        in_specs=[
            pl.BlockSpec((_BN, FW3), lambda i: (i, 0)),
            pl.BlockSpec((_BN, FW3), lambda i: (i, 0)),
            pl.BlockSpec((1, FW3), lambda i: (0, 0)),
        ],
        out_specs=pl.BlockSpec((_BN, FW3), lambda i: (i, 0)),
        out_shape=jax.ShapeDtypeStruct((NP, FW3), f32),
    )(p0, p1, brow)


def _make_sca():
    mesh = plsc.VectorSubcoreMesh(core_axis_name="c", subcore_axis_name="s")

    @functools.partial(
        pl.kernel,
        out_type=[
            jax.ShapeDtypeStruct((EP, HL), f32),
            jax.ShapeDtypeStruct((2, NP, HL), f32),
        ],
        scratch_types=[
            pltpu.VMEM((NCHUNK, CHUNK), jnp.int32),
            pltpu.VMEM((NCHUNK, CHUNK), jnp.int32),
            pltpu.VMEM((2, CHUNK, HL), f32),
            pltpu.VMEM((2, CHUNK, HL), f32),
            pltpu.VMEM((2, CHUNK, HL), f32),
            pltpu.VMEM((RPT, HL), f32),
            pltpu.VMEM_SHARED((NP, HL), f32),
            pltpu.SemaphoreType.DMA,
            pltpu.SemaphoreType.DMA,
            pltpu.SemaphoreType.DMA,
            pltpu.SemaphoreType.DMA,
            pltpu.SemaphoreType.DMA,
            pltpu.SemaphoreType.DMA,
            pltpu.SemaphoreType.DMA,
            pltpu.SemaphoreType.DMA,
        ],
        mesh=mesh,
        compiler_params=pltpu.CompilerParams(use_tc_tiling_on_sc=False),
    )
    def sca(srcp, dstp, alsrc, aldst, ex_out, den_out,
            idx_s, idx_d, asv, adv, exv, dumpv, den_sp,
            sas0, sas1, sad0, sad1, sst0, sst1, ssc0, ssc1):
        c = lax.axis_index("c")
        s = lax.axis_index("s")
        wid = c * 16 + s
        base_e = wid * EPT
        sas = (sas0, sas1)
        sad = (sad0, sad1)
        sst = (sst0, sst1)
        ssc = (ssc0, ssc1)

        pltpu.sync_copy(srcp.at[pl.ds(wid * NCHUNK, NCHUNK)], idx_s)
        pltpu.sync_copy(dstp.at[pl.ds(wid * NCHUNK, NCHUNK)], idx_d)

        def prefetch(i, b):
            pltpu.async_copy(alsrc.at[idx_s.at[i]], asv.at[b], sas[b])
            pltpu.async_copy(aldst.at[idx_d.at[i]], adv.at[b], sad[b])

        prefetch(0, 0)
        for r in range(CHUNK):
            dumpv[r] = jnp.zeros((HL,), f32)
        row0 = s * RPT
        for k in range(RPT // CHUNK):
            pltpu.sync_copy(dumpv.at[pl.ds(0, CHUNK)],
                            den_sp.at[pl.ds(row0 + k * CHUNK, CHUNK)])
        plsc.subcore_barrier()

        def body(kk, carry):
            for b in range(2):
                i = kk * 2 + b
                nb = 1 - b

                @pl.when(i < NCHUNK - 1)
                def _():
                    prefetch(i + 1, nb)

                pltpu.make_async_copy(
                    alsrc.at[pl.ds(0, CHUNK)], asv.at[b], sas[b]).wait()
                pltpu.make_async_copy(
                    aldst.at[pl.ds(0, CHUNK)], adv.at[b], sad[b]).wait()

                @pl.when(i >= 2)
                def _():
                    pltpu.make_async_copy(
                        ex_out.at[pl.ds(0, CHUNK)], exv.at[b],
                        sst[b]).wait()

                def cbody(j, c2):
                    for u in range(8):
                        r = j * 8 + u
                        v = asv[b, r] + adv[b, r]
                        v = jnp.maximum(v, 0.2 * v)
                        exv[b, r] = jnp.exp(v)
                    return c2

                lax.fori_loop(0, CHUNK // 8, cbody, 0)
                off = base_e + i * CHUNK
                pltpu.async_copy(exv.at[b], ex_out.at[pl.ds(off, CHUNK)],
                                 sst[b])
                pltpu.sync_copy(exv.at[b], den_sp.at[idx_d.at[i]],
                                add=True)
            return carry

        lax.fori_loop(0, NCHUNK // 2, body, 0)
        for b in range(2):
            pltpu.make_async_copy(
                ex_out.at[pl.ds(0, CHUNK)], exv.at[b], sst[b]).wait()
        plsc.subcore_barrier()
        pltpu.sync_copy(den_sp.at[pl.ds(row0, RPT)], dumpv)
        pltpu.sync_copy(dumpv, den_out.at[c].at[pl.ds(row0, RPT)])

    return sca


def _make_scb(fw):
    nseg = fw // 16
    mesh = plsc.VectorSubcoreMesh(core_axis_name="c", subcore_axis_name="s")

    @functools.partial(
        pl.kernel,
        out_type=[jax.ShapeDtypeStruct((2, NP, fw), f32)],
        scratch_types=[
            pltpu.VMEM((NCHUNK, CHUNK), jnp.int32),
            pltpu.VMEM((NCHUNK, CHUNK), jnp.int32),
            pltpu.VMEM((2, CHUNK, fw), f32),
            pltpu.VMEM((2, CHUNK, HL), f32),
            pltpu.VMEM((2, CHUNK, HL), f32),
            pltpu.VMEM((2, CHUNK, HL), f32),
            pltpu.VMEM((2, CHUNK, fw), f32),
            pltpu.VMEM_SHARED((NP, fw), f32),
            pltpu.SemaphoreType.DMA,
            pltpu.SemaphoreType.DMA,
            pltpu.SemaphoreType.DMA,
            pltpu.SemaphoreType.DMA,
            pltpu.SemaphoreType.DMA,
            pltpu.SemaphoreType.DMA,
            pltpu.SemaphoreType.DMA,
            pltpu.SemaphoreType.DMA,
            pltpu.SemaphoreType.DMA,
            pltpu.SemaphoreType.DMA,
        ],
        mesh=mesh,
        compiler_params=pltpu.CompilerParams(use_tc_tiling_on_sc=False),
    )
    def scb(srcp, dstp, ex_in, den0, den1, h_in, out_hbm,
            idx_s, idx_d, hv, d0v, d1v, exv, msgv, out_sp,
            sh0, sh1, s00, s01, s10, s11, se0, se1, ssc0, ssc1):
        c = lax.axis_index("c")
        s = lax.axis_index("s")
        wid = c * 16 + s
        base_e = wid * EPT
        sh = (sh0, sh1)
        s0 = (s00, s01)
        s1 = (s10, s11)
        se = (se0, se1)
        ssc = (ssc0, ssc1)

        pltpu.sync_copy(srcp.at[pl.ds(wid * NCHUNK, NCHUNK)], idx_s)
        pltpu.sync_copy(dstp.at[pl.ds(wid * NCHUNK, NCHUNK)], idx_d)

        def prefetch(i, b):
            off = base_e + i * CHUNK
            pltpu.async_copy(h_in.at[idx_s.at[i]], hv.at[b], sh[b])
            pltpu.async_copy(den0.at[idx_d.at[i]], d0v.at[b], s0[b])
            pltpu.async_copy(den1.at[idx_d.at[i]], d1v.at[b], s1[b])
            pltpu.async_copy(ex_in.at[pl.ds(off, CHUNK)], exv.at[b], se[b])

        prefetch(0, 0)
        for r in range(CHUNK):
            for v in range(nseg):
                msgv[1, r, pl.ds(v * 16, 16)] = jnp.zeros((16,), f32)
        row0 = s * RPT
        for k in range(RPT // CHUNK):
            pltpu.sync_copy(msgv.at[1],
                            out_sp.at[pl.ds(row0 + k * CHUNK, CHUNK)])
        plsc.subcore_barrier()

        lo_mask = lax.iota(jnp.int32, 16) < 8

        def body(kk, carry):
            for b in range(2):
                i = kk * 2 + b
                nb = 1 - b

                @pl.when(i < NCHUNK - 1)
                def _():
                    prefetch(i + 1, nb)

                pltpu.make_async_copy(
                    h_in.at[pl.ds(0, CHUNK)], hv.at[b], sh[b]).wait()
                pltpu.make_async_copy(
                    den0.at[pl.ds(0, CHUNK)], d0v.at[b], s0[b]).wait()
                pltpu.make_async_copy(
                    den1.at[pl.ds(0, CHUNK)], d1v.at[b], s1[b]).wait()
                pltpu.make_async_copy(
                    ex_in.at[pl.ds(0, CHUNK)], exv.at[b], se[b]).wait()

                def rbody(j, c2):
                    for u in range(8):
                        r = j * 8 + u
                        exv[b, r] = exv[b, r] / (
                            d0v[b, r] + d1v[b, r] + 1e-16)
                    return c2

                lax.fori_loop(0, CHUNK // 8, rbody, 0)

                def mbody(j, c2):
                    for u in range(4):
                        e = j * 4 + u
                        arow = exv[b, e]
                        for v in range(nseg):
                            seg = pl.ds(v * 16, 16)
                            if fw == HID:
                                a16 = jnp.where(lo_mask, arow[2 * v],
                                                arow[2 * v + 1])
                            else:
                                a16 = jnp.where(lo_mask, arow[0], arow[0])
                            msgv[b, e, seg] = hv[b, e, seg] * a16
                    return c2

                lax.fori_loop(0, CHUNK // 4, mbody, 0)
                pltpu.sync_copy(msgv.at[b], out_sp.at[idx_d.at[i]],
                                add=True)
            return carry

        lax.fori_loop(0, NCHUNK // 2, body, 0)
        plsc.subcore_barrier()
        for k in range(RPT // CHUNK):
            rr = row0 + k * CHUNK
            pltpu.sync_copy(out_sp.at[pl.ds(rr, CHUNK)], msgv.at[k % 2])
            pltpu.sync_copy(msgv.at[k % 2],
                            out_hbm.at[c].at[pl.ds(rr, CHUNK)])

    return scb


_SCA = _make_sca()
_SCB64 = _make_scb(HID)
_SCB48 = _make_scb(FW3)


def _expand_att(a):
    ar = jnp.arange(HID)
    return jnp.zeros((HID, HL), f32).at[ar, ar // DIM].set(a.reshape(HID))


def kernel(x, edge_index, W1, a_src1, a_dst1, b1, W2, a_src2, a_dst2, b2,
           W3, a_src3, a_dst3, b3):
    xp = jnp.zeros((NP, D_IN), f32).at[:N].set(x)
    eper = E // NTILES
    src = edge_index[0].reshape(NTILES, eper)
    dst = edge_index[1].reshape(NTILES, eper)
    padw = ((0, 0), (0, EPT - eper))
    srcp = jnp.pad(src, padw, constant_values=TRASH).reshape(
        NTILES * NCHUNK, CHUNK)
    dstp = jnp.pad(dst, padw, constant_values=TRASH).reshape(
        NTILES * NCHUNK, CHUNK)

    As1, Ad1 = _expand_att(a_src1), _expand_att(a_dst1)
    As2, Ad2 = _expand_att(a_src2), _expand_att(a_dst2)
    a3s = jnp.zeros((FW3,), f32).at[:NCLS].set(a_src3[0])
    a3d = jnp.zeros((FW3,), f32).at[:NCLS].set(a_dst3[0])
    As3 = jnp.tile(a3s[:, None], (1, HL))
    Ad3 = jnp.tile(a3d[:, None], (1, HL))
    W3p = jnp.zeros((HID, FW3), f32).at[:, :NCLS].set(W3)
    b3p = jnp.zeros((FW3,), f32).at[:NCLS].set(b3)

    h1, as1, ad1 = _dense1(xp, W1, As1, Ad1)
    ex1, den1 = _SCA(srcp, dstp, as1, ad1)
    (outp1,) = _SCB64(srcp, dstp, ex1, den1[0], den1[1], h1)
    h2, as2, ad2 = _dense_mid(outp1[0], outp1[1], b1.reshape(1, HID),
                              W2, As2, Ad2)
    ex2, den2 = _SCA(srcp, dstp, as2, ad2)
    (outp2,) = _SCB64(srcp, dstp, ex2, den2[0], den2[1], h2)
    h3, as3, ad3 = _dense_mid(outp2[0], outp2[1], b2.reshape(1, HID),
                              W3p, As3, Ad3)
    ex3, den3 = _SCA(srcp, dstp, as3, ad3)
    (outp3,) = _SCB48(srcp, dstp, ex3, den3[0], den3[1], h3)
    o = _final(outp3[0], outp3[1], b3p.reshape(1, FW3))
    return o[:N, :NCLS]


# speedup vs baseline: 71.3996x; 1.0897x over previous
import functools

import jax
import jax.numpy as jnp
from jax import lax
from jax.experimental import pallas as pl
from jax.experimental.pallas import tpu as pltpu
from jax.experimental.pallas import tpu_sc as plsc

N = 10000
E = 320000
D_IN = 128
HEADS = 8
DIM = 8
HID = 64
NCLS = 40
FW3 = 48
NP = 10240
HL = 16
CHUNK = 128
NTILES = 32
EPT = 10240
NCHUNK = EPT // CHUNK
EP = NTILES * EPT
TRASH = NP - 1
RPT = NP // 16

f32 = jnp.float32
_BN = 256


def _dense1_body(x_ref, w_ref, asw_ref, adw_ref, h_ref, as_ref, ad_ref):
    h = jnp.dot(x_ref[...], w_ref[...], preferred_element_type=f32)
    h_ref[...] = h
    as_ref[...] = jnp.dot(h, asw_ref[...], preferred_element_type=f32)
    ad_ref[...] = jnp.dot(h, adw_ref[...], preferred_element_type=f32)


def _dense1(x, w, asw, adw):
    return pl.pallas_call(
        _dense1_body,
        grid=(NP // _BN,),
        in_specs=[
            pl.BlockSpec((_BN, D_IN), lambda i: (i, 0)),
            pl.BlockSpec((D_IN, HID), lambda i: (0, 0)),
            pl.BlockSpec((HID, HL), lambda i: (0, 0)),
            pl.BlockSpec((HID, HL), lambda i: (0, 0)),
        ],
        out_specs=[
            pl.BlockSpec((_BN, HID), lambda i: (i, 0)),
            pl.BlockSpec((_BN, HL), lambda i: (i, 0)),
            pl.BlockSpec((_BN, HL), lambda i: (i, 0)),
        ],
        out_shape=[
            jax.ShapeDtypeStruct((NP, HID), f32),
            jax.ShapeDtypeStruct((NP, HL), f32),
            jax.ShapeDtypeStruct((NP, HL), f32),
        ],
    )(x, w, asw, adw)


def _dense_mid_body(p0_ref, p1_ref, b_ref, w_ref, asw_ref, adw_ref,
                    h_ref, as_ref, ad_ref):
    o = p0_ref[...] + p1_ref[...] + b_ref[...]
    e = jnp.where(o > 0, o, jnp.exp(jnp.minimum(o, 0.0)) - 1.0)
    h = jnp.dot(e, w_ref[...], preferred_element_type=f32)
    h_ref[...] = h
    as_ref[...] = jnp.dot(h, asw_ref[...], preferred_element_type=f32)
    ad_ref[...] = jnp.dot(h, adw_ref[...], preferred_element_type=f32)


def _dense_mid(p0, p1, brow, w, asw, adw):
    fo = w.shape[1]
    return pl.pallas_call(
        _dense_mid_body,
        grid=(NP // _BN,),
        in_specs=[
            pl.BlockSpec((_BN, HID), lambda i: (i, 0)),
            pl.BlockSpec((_BN, HID), lambda i: (i, 0)),
            pl.BlockSpec((1, HID), lambda i: (0, 0)),
            pl.BlockSpec((HID, fo), lambda i: (0, 0)),
            pl.BlockSpec((fo, HL), lambda i: (0, 0)),
            pl.BlockSpec((fo, HL), lambda i: (0, 0)),
        ],
        out_specs=[
            pl.BlockSpec((_BN, fo), lambda i: (i, 0)),
            pl.BlockSpec((_BN, HL), lambda i: (i, 0)),
            pl.BlockSpec((_BN, HL), lambda i: (i, 0)),
        ],
        out_shape=[
            jax.ShapeDtypeStruct((NP, fo), f32),
            jax.ShapeDtypeStruct((NP, HL), f32),
            jax.ShapeDtypeStruct((NP, HL), f32),
        ],
    )(p0, p1, brow, w, asw, adw)


def _comb_body(a_ref, b_ref, o_ref):
    o_ref[...] = a_ref[...] + b_ref[...] + 1e-16


def _comb(a, b):
    nr = a.shape[0]
    return pl.pallas_call(
        _comb_body,
        grid=(nr // 256,),
        in_specs=[
            pl.BlockSpec((256, 128), lambda i: (i, 0)),
            pl.BlockSpec((256, 128), lambda i: (i, 0)),
        ],
        out_specs=pl.BlockSpec((256, 128), lambda i: (i, 0)),
        out_shape=jax.ShapeDtypeStruct((nr, 128), f32),
    )(a, b)


def _final_body(p0_ref, p1_ref, b_ref, o_ref):
    z = p0_ref[...] + p1_ref[...] + b_ref[...]
    col = lax.broadcasted_iota(jnp.int32, z.shape, 1)
    mask = col < NCLS
    zm = jnp.where(mask, z, -1e30)
    m = jnp.max(zm, axis=1, keepdims=True)
    ez = jnp.where(mask, jnp.exp(z - m), 0.0)
    ssum = jnp.sum(ez, axis=1, keepdims=True)
    o_ref[...] = z - m - jnp.log(ssum)


def _final(p0, p1, brow):
    return pl.pallas_call(
        _final_body,
        grid=(NP // _BN,),
        in_specs=[
            pl.BlockSpec((_BN, FW3), lambda i: (i, 0)),
            pl.BlockSpec((_BN, FW3), lambda i: (i, 0)),
            pl.BlockSpec((1, FW3), lambda i: (0, 0)),
        ],
        out_specs=pl.BlockSpec((_BN, FW3), lambda i: (i, 0)),
        out_shape=jax.ShapeDtypeStruct((NP, FW3), f32),
    )(p0, p1, brow)


def _make_sca():
    mesh = plsc.VectorSubcoreMesh(core_axis_name="c", subcore_axis_name="s")

    @functools.partial(
        pl.kernel,
        out_type=[
            jax.ShapeDtypeStruct((EP, HL), f32),
            jax.ShapeDtypeStruct((2, NP, HL), f32),
        ],
        scratch_types=[
            pltpu.VMEM((NCHUNK, CHUNK), jnp.int32),
            pltpu.VMEM((NCHUNK, CHUNK), jnp.int32),
            pltpu.VMEM((2, CHUNK, HL), f32),
            pltpu.VMEM((2, CHUNK, HL), f32),
            pltpu.VMEM((2, CHUNK, HL), f32),
            pltpu.VMEM((RPT, HL), f32),
            pltpu.VMEM_SHARED((NP, HL), f32),
            pltpu.SemaphoreType.DMA,
            pltpu.SemaphoreType.DMA,
            pltpu.SemaphoreType.DMA,
            pltpu.SemaphoreType.DMA,
            pltpu.SemaphoreType.DMA,
            pltpu.SemaphoreType.DMA,
            pltpu.SemaphoreType.DMA,
            pltpu.SemaphoreType.DMA,
        ],
        mesh=mesh,
        compiler_params=pltpu.CompilerParams(use_tc_tiling_on_sc=False),
    )
    def sca(srcp, dstp, alsrc, aldst, ex_out, den_out,
            idx_s, idx_d, asv, adv, exv, dumpv, den_sp,
            sas0, sas1, sad0, sad1, sst0, sst1, ssc0, ssc1):
        c = lax.axis_index("c")
        s = lax.axis_index("s")
        wid = c * 16 + s
        base_e = wid * EPT
        sas = (sas0, sas1)
        sad = (sad0, sad1)
        sst = (sst0, sst1)
        ssc = (ssc0, ssc1)

        pltpu.sync_copy(srcp.at[pl.ds(wid * NCHUNK, NCHUNK)], idx_s)
        pltpu.sync_copy(dstp.at[pl.ds(wid * NCHUNK, NCHUNK)], idx_d)

        def prefetch(i, b):
            pltpu.async_copy(alsrc.at[idx_s.at[i]], asv.at[b], sas[b])
            pltpu.async_copy(aldst.at[idx_d.at[i]], adv.at[b], sad[b])

        prefetch(0, 0)
        for r in range(CHUNK):
            dumpv[r] = jnp.zeros((HL,), f32)
        row0 = s * RPT
        for k in range(RPT // CHUNK):
            pltpu.sync_copy(dumpv.at[pl.ds(0, CHUNK)],
                            den_sp.at[pl.ds(row0 + k * CHUNK, CHUNK)])
        plsc.subcore_barrier()

        def body(kk, carry):
            for b in range(2):
                i = kk * 2 + b
                nb = 1 - b

                @pl.when(i < NCHUNK - 1)
                def _():
                    prefetch(i + 1, nb)

                pltpu.make_async_copy(
                    alsrc.at[pl.ds(0, CHUNK)], asv.at[b], sas[b]).wait()
                pltpu.make_async_copy(
                    aldst.at[pl.ds(0, CHUNK)], adv.at[b], sad[b]).wait()

                @pl.when(i >= 2)
                def _():
                    pltpu.make_async_copy(
                        ex_out.at[pl.ds(0, CHUNK)], exv.at[b],
                        sst[b]).wait()
                    pltpu.make_async_copy(
                        ex_out.at[pl.ds(0, CHUNK)], exv.at[b],
                        ssc[b]).wait()

                def cbody(j, c2):
                    for u in range(8):
                        r = j * 8 + u
                        v = asv[b, r] + adv[b, r]
                        v = jnp.maximum(v, 0.2 * v)
                        exv[b, r] = jnp.exp(v)
                    return c2

                lax.fori_loop(0, CHUNK // 8, cbody, 0)
                off = base_e + i * CHUNK
                pltpu.async_copy(exv.at[b], ex_out.at[pl.ds(off, CHUNK)],
                                 sst[b])
                pltpu.async_copy(exv.at[b], den_sp.at[idx_d.at[i]],
                                 ssc[b], add=True)
            return carry

        lax.fori_loop(0, NCHUNK // 2, body, 0)
        for b in range(2):
            pltpu.make_async_copy(
                ex_out.at[pl.ds(0, CHUNK)], exv.at[b], sst[b]).wait()
            pltpu.make_async_copy(
                ex_out.at[pl.ds(0, CHUNK)], exv.at[b], ssc[b]).wait()
        plsc.subcore_barrier()
        pltpu.sync_copy(den_sp.at[pl.ds(row0, RPT)], dumpv)
        pltpu.sync_copy(dumpv, den_out.at[c].at[pl.ds(row0, RPT)])

    return sca


def _make_scb(fw):
    nseg = fw // 16
    mesh = plsc.VectorSubcoreMesh(core_axis_name="c", subcore_axis_name="s")

    @functools.partial(
        pl.kernel,
        out_type=[jax.ShapeDtypeStruct((2, NP, fw), f32)],
        scratch_types=[
            pltpu.VMEM((NCHUNK, CHUNK), jnp.int32),
            pltpu.VMEM((NCHUNK, CHUNK), jnp.int32),
            pltpu.VMEM((2, CHUNK, fw), f32),
            pltpu.VMEM((2, CHUNK, HL), f32),
            pltpu.VMEM((2, CHUNK, HL), f32),
            pltpu.VMEM((2, CHUNK, fw), f32),
            pltpu.VMEM_SHARED((NP, fw), f32),
            pltpu.SemaphoreType.DMA,
            pltpu.SemaphoreType.DMA,
            pltpu.SemaphoreType.DMA,
            pltpu.SemaphoreType.DMA,
            pltpu.SemaphoreType.DMA,
            pltpu.SemaphoreType.DMA,
            pltpu.SemaphoreType.DMA,
            pltpu.SemaphoreType.DMA,
        ],
        mesh=mesh,
        compiler_params=pltpu.CompilerParams(use_tc_tiling_on_sc=False),
    )
    def scb(srcp, dstp, ex_in, den, h_in, out_hbm,
            idx_s, idx_d, hv, d0v, exv, msgv, out_sp,
            sh0, sh1, s00, s01, se0, se1, ssc0, ssc1):
        c = lax.axis_index("c")
        s = lax.axis_index("s")
        wid = c * 16 + s
        base_e = wid * EPT
        sh = (sh0, sh1)
        s0 = (s00, s01)
        se = (se0, se1)
        ssc = (ssc0, ssc1)

        pltpu.sync_copy(srcp.at[pl.ds(wid * NCHUNK, NCHUNK)], idx_s)
        pltpu.sync_copy(dstp.at[pl.ds(wid * NCHUNK, NCHUNK)], idx_d)

        def prefetch(i, b):
            off = base_e + i * CHUNK
            pltpu.async_copy(h_in.at[idx_s.at[i]], hv.at[b], sh[b])
            pltpu.async_copy(den.at[idx_d.at[i]], d0v.at[b], s0[b])
            pltpu.async_copy(ex_in.at[pl.ds(off, CHUNK)], exv.at[b], se[b])

        prefetch(0, 0)
        for r in range(CHUNK):
            for v in range(nseg):
                msgv[1, r, pl.ds(v * 16, 16)] = jnp.zeros((16,), f32)
        row0 = s * RPT
        for k in range(RPT // CHUNK):
            pltpu.sync_copy(msgv.at[1],
                            out_sp.at[pl.ds(row0 + k * CHUNK, CHUNK)])
        plsc.subcore_barrier()

        lo_mask = lax.iota(jnp.int32, 16) < 8

        def body(kk, carry):
            for b in range(2):
                i = kk * 2 + b
                nb = 1 - b

                @pl.when(i < NCHUNK - 1)
                def _():
                    prefetch(i + 1, nb)

                pltpu.make_async_copy(
                    h_in.at[pl.ds(0, CHUNK)], hv.at[b], sh[b]).wait()
                pltpu.make_async_copy(
                    den.at[pl.ds(0, CHUNK)], d0v.at[b], s0[b]).wait()
                pltpu.make_async_copy(
                    ex_in.at[pl.ds(0, CHUNK)], exv.at[b], se[b]).wait()

                @pl.when(i >= 2)
                def _():
                    pltpu.make_async_copy(
                        h_in.at[pl.ds(0, CHUNK)], msgv.at[b],
                        ssc[b]).wait()

                def rbody(j, c2):
                    for u in range(8):
                        r = j * 8 + u
                        exv[b, r] = exv[b, r] / d0v[b, r]
                    return c2

                lax.fori_loop(0, CHUNK // 8, rbody, 0)

                def mbody(j, c2):
                    for u in range(4):
                        e = j * 4 + u
                        arow = exv[b, e]
                        for v in range(nseg):
                            seg = pl.ds(v * 16, 16)
                            if fw == HID:
                                a16 = jnp.where(lo_mask, arow[2 * v],
                                                arow[2 * v + 1])
                            else:
                                a16 = jnp.where(lo_mask, arow[0], arow[0])
                            msgv[b, e, seg] = hv[b, e, seg] * a16
                    return c2

                lax.fori_loop(0, CHUNK // 4, mbody, 0)
                pltpu.async_copy(msgv.at[b], out_sp.at[idx_d.at[i]],
                                 ssc[b], add=True)
            return carry

        lax.fori_loop(0, NCHUNK // 2, body, 0)
        for b in range(2):
            pltpu.make_async_copy(
                h_in.at[pl.ds(0, CHUNK)], msgv.at[b], ssc[b]).wait()
        plsc.subcore_barrier()
        for k in range(RPT // CHUNK):
            rr = row0 + k * CHUNK
            pltpu.sync_copy(out_sp.at[pl.ds(rr, CHUNK)], msgv.at[k % 2])
            pltpu.sync_copy(msgv.at[k % 2],
                            out_hbm.at[c].at[pl.ds(rr, CHUNK)])

    return scb


_SCA = _make_sca()
_SCB64 = _make_scb(HID)
_SCB48 = _make_scb(FW3)


def _expand_att(a):
    ar = jnp.arange(HID)
    return jnp.zeros((HID, HL), f32).at[ar, ar // DIM].set(a.reshape(HID))


def kernel(x, edge_index, W1, a_src1, a_dst1, b1, W2, a_src2, a_dst2, b2,
           W3, a_src3, a_dst3, b3):
    xp = jnp.zeros((NP, D_IN), f32).at[:N].set(x)
    eper = E // NTILES
    src = edge_index[0].reshape(NTILES, eper)
    dst = edge_index[1].reshape(NTILES, eper)
    padw = ((0, 0), (0, EPT - eper))
    srcp = jnp.pad(src, padw, constant_values=TRASH).reshape(
        NTILES * NCHUNK, CHUNK)
    dstp = jnp.pad(dst, padw, constant_values=TRASH).reshape(
        NTILES * NCHUNK, CHUNK)

    As1, Ad1 = _expand_att(a_src1), _expand_att(a_dst1)
    As2, Ad2 = _expand_att(a_src2), _expand_att(a_dst2)
    a3s = jnp.zeros((FW3,), f32).at[:NCLS].set(a_src3[0])
    a3d = jnp.zeros((FW3,), f32).at[:NCLS].set(a_dst3[0])
    As3 = jnp.tile(a3s[:, None], (1, HL))
    Ad3 = jnp.tile(a3d[:, None], (1, HL))
    W3p = jnp.zeros((HID, FW3), f32).at[:, :NCLS].set(W3)
    b3p = jnp.zeros((FW3,), f32).at[:NCLS].set(b3)

    def comb(den):
        d2 = den.reshape(2, NP * HL // 128, 128)
        return _comb(d2[0], d2[1]).reshape(NP, HL)

    h1, as1, ad1 = _dense1(xp, W1, As1, Ad1)
    ex1, den1 = _SCA(srcp, dstp, as1, ad1)
    (outp1,) = _SCB64(srcp, dstp, ex1, comb(den1), h1)
    h2, as2, ad2 = _dense_mid(outp1[0], outp1[1], b1.reshape(1, HID),
                              W2, As2, Ad2)
    ex2, den2 = _SCA(srcp, dstp, as2, ad2)
    (outp2,) = _SCB64(srcp, dstp, ex2, comb(den2), h2)
    h3, as3, ad3 = _dense_mid(outp2[0], outp2[1], b2.reshape(1, HID),
                              W3p, As3, Ad3)
    ex3, den3 = _SCA(srcp, dstp, as3, ad3)
    (outp3,) = _SCB48(srcp, dstp, ex3, comb(den3), h3)
    o = _final(outp3[0], outp3[1], b3p.reshape(1, FW3))
    return o[:N, :NCLS]
